# Initial kernel scaffold; baseline (speedup 1.0000x reference)
import jax
import jax.numpy as jnp
from jax.experimental import pallas as pl


def kernel(x):
    raise NotImplementedError("write your pallas kernel here")


# speedup vs baseline: 2.5373x; 2.5373x over previous
import math

import jax
import jax.numpy as jnp
import numpy as np
from jax.experimental import pallas as pl
from jax.experimental.pallas import tpu as pltpu

_T = 64
_N = 4096
_C = 128
_BN = 256


def _sg_endpoint_weights() -> np.ndarray:
    K, p, m, dt = _T, 2, 1, 0.02
    x = jnp.arange(-K + 1, 1, dtype=jnp.float32) * float(dt)
    A = jnp.stack([x**j for j in range(p + 1)], axis=1)
    ATA_pinv = jnp.linalg.pinv(A.T @ A)
    e_m = jnp.zeros(p + 1, dtype=jnp.float32).at[m].set(1.0)
    w = (e_m @ ATA_pinv @ A.T) * float(math.factorial(m))
    return np.asarray(w, dtype=np.float32)


_W = _sg_endpoint_weights()


def _body(x_ref, o_ref):
    acc = _W[0] * x_ref[0]
    for t in range(1, _T):
        acc = acc + _W[t] * x_ref[t]
    o_ref[...] = acc


def kernel(x):
    return pl.pallas_call(
        _body,
        grid=(_N // _BN,),
        in_specs=[pl.BlockSpec((_T, _BN, _C), lambda i: (0, i, 0))],
        out_specs=pl.BlockSpec((_BN, _C), lambda i: (i, 0)),
        out_shape=jax.ShapeDtypeStruct((_N, _C), jnp.float32),
        compiler_params=pltpu.CompilerParams(
            dimension_semantics=("arbitrary",),
        ),
    )(x)
